# baseline (speedup 1.0000x reference)
import jax
import jax.numpy as jnp
from jax.experimental import pallas as pl
from jax.experimental.pallas import tpu as pltpu

N = 2048
N2 = N // 2
D_IN = 512
B_HID = 4096
CODE = 64
NUM_GRIDS = 8
GRID_MIN, GRID_MAX = -2.0, 2.0
D_OUT = 2 * D_IN
KAN_K = CODE * NUM_GRIDS

BM = 512
NA = N // BM
ND = N // BM
T_TOTAL = NA + ND
W3_CHUNK = D_OUT // NA

_DENOM = (GRID_MAX - GRID_MIN) / (NUM_GRIDS - 1)


def _dot(a, b):
    return jax.lax.dot_general(
        a, b, (((1,), (0,)), ((), ())),
        preferred_element_type=jnp.float32)


def _bf16(v):
    return v.astype(jnp.bfloat16)


def _fused_kernel(gl_ref, gr_ref, x_ref, w1_ref, b1_ref, w2_ref, b2_ref,
                  lnw_ref, lnb_ref, w3_ref, b3_ref,
                  code_ref, out_ref,
                  g_scr, xb_scr, w1b_scr, w2b_scr, w3b_scr, t2_scr, t2b_scr):
    t = pl.program_id(0)

    @pl.when(t == 0)
    def _pack_resident():
        xb_scr[...] = _bf16(x_ref[...])
        w1b_scr[...] = _bf16(w1_ref[...])
        w2b_scr[...] = _bf16(w2_ref[...])

    @pl.when(t < NA)
    def _phase_a():
        i = t
        w3b_scr[:, pl.ds(i * W3_CHUNK, W3_CHUNK)] = \
            _bf16(w3_ref[:, pl.ds(i * W3_CHUNK, W3_CHUNK)])

        glb = _bf16(gl_ref[...])
        grb = _bf16(gr_ref[...])
        g_scr[pl.ds(i * BM, BM), 0:N2] = glb
        g_scr[pl.ds(i * BM, BM), N2:N] = grb
        y = _dot(glb, xb_scr[0:N2, :]) + _dot(grb, xb_scr[N2:N, :])
        h = jnp.maximum(_dot(_bf16(y), w1b_scr[...]) + b1_ref[...], 0.0)
        t2_scr[pl.ds(i * BM, BM), :] = _dot(_bf16(h), w2b_scr[...])

    @pl.when(t >= NA)
    def _phase_d():
        i = t - NA

        @pl.when(i == 0)
        def _pack_t2():
            t2b_scr[...] = _bf16(t2_scr[...])

        feat = _dot(g_scr[pl.ds(i * BM, BM), :], t2b_scr[...]) + b2_ref[...]
        code = jnp.tanh(10.0 * feat)
        code_ref[...] = code

        mu = jnp.mean(code, axis=-1, keepdims=True)
        var = jnp.mean((code - mu) ** 2, axis=-1, keepdims=True)
        y = (code - mu) * jax.lax.rsqrt(var + 1e-5) * lnw_ref[...] + lnb_ref[...]

        yt = jnp.tile(y, (1, NUM_GRIDS))
        gidx = jax.lax.broadcasted_iota(jnp.int32, (1, KAN_K), 1) // CODE
        gcols = GRID_MIN + gidx.astype(jnp.float32) * _DENOM
        tt = (yt - gcols) * (1.0 / _DENOM)
        rbf = jnp.exp(-(tt * tt))
        acc = _dot(_bf16(rbf), w3b_scr[...])
        out_ref[...] = jnp.maximum(acc + b3_ref[...], 0.0)


@jax.jit
def kernel(x, G, W1, b1, W2, b2, ln_w, ln_b, W3, b3):
    W3p = W3.reshape(CODE, NUM_GRIDS, D_OUT).transpose(1, 0, 2).reshape(KAN_K, D_OUT)
    row = lambda v: v.reshape(1, -1)

    full = lambda shape: pl.BlockSpec(shape, lambda t: (0,) * len(shape))

    code, feat_out = pl.pallas_call(
        _fused_kernel,
        grid=(T_TOTAL,),
        in_specs=[
            pl.BlockSpec((BM, N2), lambda t: (jnp.minimum(t, NA - 1), 0)),
            pl.BlockSpec((BM, N2), lambda t: (jnp.minimum(t, NA - 1), 1)),
            full((N, D_IN)),
            full((D_IN, B_HID)),
            full((1, B_HID)),
            full((B_HID, CODE)),
            full((1, CODE)),
            full((1, CODE)),
            full((1, CODE)),
            full((KAN_K, D_OUT)),
            full((1, D_OUT)),
        ],
        out_specs=[
            pl.BlockSpec((BM, CODE), lambda t: (jnp.maximum(t - NA, 0), 0)),
            pl.BlockSpec((BM, D_OUT), lambda t: (jnp.maximum(t - NA, 0), 0)),
        ],
        out_shape=[
            jax.ShapeDtypeStruct((N, CODE), jnp.float32),
            jax.ShapeDtypeStruct((N, D_OUT), jnp.float32),
        ],
        scratch_shapes=[
            pltpu.VMEM((N, N), jnp.bfloat16),
            pltpu.VMEM((N, D_IN), jnp.bfloat16),
            pltpu.VMEM((D_IN, B_HID), jnp.bfloat16),
            pltpu.VMEM((B_HID, CODE), jnp.bfloat16),
            pltpu.VMEM((KAN_K, D_OUT), jnp.bfloat16),
            pltpu.VMEM((N, CODE), jnp.float32),
            pltpu.VMEM((N, CODE), jnp.bfloat16),
        ],
        compiler_params=pltpu.CompilerParams(
            dimension_semantics=("arbitrary",)),
    )(G, G, x, W1, row(b1), W2, row(b2), row(ln_w), row(ln_b), W3p, row(b3))
    return (code, feat_out)


# speedup vs baseline: 1.1494x; 1.1494x over previous
import jax
import jax.numpy as jnp
from jax.experimental import pallas as pl
from jax.experimental.pallas import tpu as pltpu

N = 2048
N2 = N // 2
D_IN = 512
B_HID = 4096
CODE = 64
NUM_GRIDS = 8
GRID_MIN, GRID_MAX = -2.0, 2.0
D_OUT = 2 * D_IN
KAN_K = CODE * NUM_GRIDS

BM = 256
NA = N // BM
ND = N // BM
T_TOTAL = NA + ND
W3_CHUNK = D_OUT // NA

_DENOM = (GRID_MAX - GRID_MIN) / (NUM_GRIDS - 1)


def _dot(a, b):
    return jax.lax.dot_general(
        a, b, (((1,), (0,)), ((), ())),
        preferred_element_type=jnp.float32)


def _bf16(v):
    return v.astype(jnp.bfloat16)


def _fused_kernel(gl_ref, gr_ref, xt_ref, xb_ref, w1a_ref, w1b_ref,
                  b1_ref, w2_ref, b2_ref,
                  lnw_ref, lnb_ref, w3_ref, b3_ref,
                  code_ref, out_ref,
                  g_scr, xb_scr, w1b_scr, w2b_scr, w3b_scr, t2_scr, t2b_scr):
    t = pl.program_id(0)

    @pl.when(t == 0)
    def _pack_resident():
        xb_scr[0:N2, :] = _bf16(xt_ref[...])
        xb_scr[N2:N, :] = _bf16(xb_ref[...])
        w1b_scr[:, 0:B_HID // 2] = _bf16(w1a_ref[...])
        w1b_scr[:, B_HID // 2:B_HID] = _bf16(w1b_ref[...])
        w2b_scr[...] = _bf16(w2_ref[...])

    @pl.when(t < NA)
    def _phase_a():
        i = t
        w3b_scr[:, pl.ds(i * W3_CHUNK, W3_CHUNK)] = \
            _bf16(w3_ref[:, pl.ds(i * W3_CHUNK, W3_CHUNK)])

        glb = _bf16(gl_ref[...])
        grb = _bf16(gr_ref[...])
        g_scr[pl.ds(i * BM, BM), 0:N2] = glb
        g_scr[pl.ds(i * BM, BM), N2:N] = grb
        y = _dot(glb, xb_scr[0:N2, :]) + _dot(grb, xb_scr[N2:N, :])
        h = jnp.maximum(_dot(_bf16(y), w1b_scr[...]) + b1_ref[...], 0.0)
        t2_scr[pl.ds(i * BM, BM), :] = _dot(_bf16(h), w2b_scr[...])

    @pl.when(t >= NA)
    def _phase_d():
        i = t - NA

        @pl.when(i == 0)
        def _pack_t2():
            t2b_scr[...] = _bf16(t2_scr[...])

        feat = _dot(g_scr[pl.ds(i * BM, BM), :], t2b_scr[...]) + b2_ref[...]
        code = jnp.tanh(10.0 * feat)
        code_ref[...] = code

        mu = jnp.mean(code, axis=-1, keepdims=True)
        var = jnp.mean((code - mu) ** 2, axis=-1, keepdims=True)
        y = (code - mu) * jax.lax.rsqrt(var + 1e-5) * lnw_ref[...] + lnb_ref[...]

        yt = jnp.tile(y, (1, NUM_GRIDS))
        gidx = jax.lax.broadcasted_iota(jnp.int32, (1, KAN_K), 1) // CODE
        gcols = GRID_MIN + gidx.astype(jnp.float32) * _DENOM
        tt = (yt - gcols) * (1.0 / _DENOM)
        rbf = jnp.exp(-(tt * tt))
        acc = _dot(_bf16(rbf), w3b_scr[...])
        out_ref[...] = jnp.maximum(acc + b3_ref[...], 0.0)


@jax.jit
def kernel(x, G, W1, b1, W2, b2, ln_w, ln_b, W3, b3):
    W3p = W3.reshape(CODE, NUM_GRIDS, D_OUT).transpose(1, 0, 2).reshape(KAN_K, D_OUT)
    row = lambda v: v.reshape(1, -1)

    full = lambda shape: pl.BlockSpec(shape, lambda t: (0,) * len(shape))

    code, feat_out = pl.pallas_call(
        _fused_kernel,
        grid=(T_TOTAL,),
        in_specs=[
            pl.BlockSpec((BM, N2), lambda t: (jnp.minimum(t, NA - 1), 0)),
            pl.BlockSpec((BM, N2), lambda t: (jnp.minimum(t, NA - 1), 1)),
            pl.BlockSpec((N2, D_IN), lambda t: (0, 0)),
            pl.BlockSpec((N2, D_IN), lambda t: (1, 0)),
            pl.BlockSpec((D_IN, B_HID // 2), lambda t: (0, 0)),
            pl.BlockSpec((D_IN, B_HID // 2), lambda t: (0, 1)),
            full((1, B_HID)),
            full((B_HID, CODE)),
            full((1, CODE)),
            full((1, CODE)),
            full((1, CODE)),
            full((KAN_K, D_OUT)),
            full((1, D_OUT)),
        ],
        out_specs=[
            pl.BlockSpec((BM, CODE), lambda t: (jnp.maximum(t - NA, 0), 0)),
            pl.BlockSpec((BM, D_OUT), lambda t: (jnp.maximum(t - NA, 0), 0)),
        ],
        out_shape=[
            jax.ShapeDtypeStruct((N, CODE), jnp.float32),
            jax.ShapeDtypeStruct((N, D_OUT), jnp.float32),
        ],
        scratch_shapes=[
            pltpu.VMEM((N, N), jnp.bfloat16),
            pltpu.VMEM((N, D_IN), jnp.bfloat16),
            pltpu.VMEM((D_IN, B_HID), jnp.bfloat16),
            pltpu.VMEM((B_HID, CODE), jnp.bfloat16),
            pltpu.VMEM((KAN_K, D_OUT), jnp.bfloat16),
            pltpu.VMEM((N, CODE), jnp.float32),
            pltpu.VMEM((N, CODE), jnp.bfloat16),
        ],
        compiler_params=pltpu.CompilerParams(
            dimension_semantics=("arbitrary",)),
    )(G, G, x, x, W1, W1, row(b1), W2, row(b2), row(ln_w), row(ln_b), W3p, row(b3))
    return (code, feat_out)


# speedup vs baseline: 1.1747x; 1.0220x over previous
import jax
import jax.numpy as jnp
from jax.experimental import pallas as pl
from jax.experimental.pallas import tpu as pltpu

N = 2048
N2 = N // 2
D_IN = 512
B_HID = 4096
CODE = 64
NUM_GRIDS = 8
GRID_MIN, GRID_MAX = -2.0, 2.0
D_OUT = 2 * D_IN
KAN_K = CODE * NUM_GRIDS

BM = 256
NA = N // BM
ND = N // BM
T_TOTAL = NA + ND
W3_CHUNK = D_OUT // NA

_DENOM = (GRID_MAX - GRID_MIN) / (NUM_GRIDS - 1)


def _dot(a, b):
    return jax.lax.dot_general(
        a, b, (((1,), (0,)), ((), ())),
        preferred_element_type=jnp.float32)


def _bf16(v):
    return v.astype(jnp.bfloat16)


def _fused_kernel(gl_ref, gr_ref, xt_ref, xb_ref, w1a_ref, w1b_ref,
                  b1_ref, w2_ref, b2_ref,
                  lnw_ref, lnb_ref, w3_ref, b3_ref,
                  code_ref, out_ref,
                  g_scr, xb_scr, w1b_scr, w2b_scr, w3b_scr, t2_scr, feat_scr):
    t = pl.program_id(0)

    @pl.when(t == 0)
    def _pack_resident():
        xb_scr[0:N2, :] = _bf16(xt_ref[...])
        xb_scr[N2:N, :] = _bf16(xb_ref[...])
        w1b_scr[:, 0:B_HID // 2] = _bf16(w1a_ref[...])
        w1b_scr[:, B_HID // 2:B_HID] = _bf16(w1b_ref[...])
        w2b_scr[...] = _bf16(w2_ref[...])

    @pl.when(t < NA)
    def _phase_a():
        i = t
        w3b_scr[:, pl.ds(i * W3_CHUNK, W3_CHUNK)] = \
            _bf16(w3_ref[:, pl.ds(i * W3_CHUNK, W3_CHUNK)])

        glb = _bf16(gl_ref[...])
        grb = _bf16(gr_ref[...])
        g_scr[pl.ds(i * BM, BM), 0:N2] = glb
        g_scr[pl.ds(i * BM, BM), N2:N] = grb
        y = _dot(glb, xb_scr[0:N2, :]) + _dot(grb, xb_scr[N2:N, :])
        h = jnp.maximum(_dot(_bf16(y), w1b_scr[...]) + b1_ref[...], 0.0)
        t2_scr[pl.ds(i * BM, BM), :] = _dot(_bf16(h), w2b_scr[...])

    @pl.when(t >= NA)
    def _phase_d():
        i = t - NA

        @pl.when(i == 0)
        def _feat_all():
            t2b = _bf16(t2_scr[...])
            feat_scr[...] = _dot(g_scr[...], t2b)

        feat = feat_scr[pl.ds(i * BM, BM), :] + b2_ref[...]
        code = jnp.tanh(10.0 * feat)
        code_ref[...] = code

        mu = jnp.mean(code, axis=-1, keepdims=True)
        var = jnp.mean((code - mu) ** 2, axis=-1, keepdims=True)
        y = (code - mu) * jax.lax.rsqrt(var + 1e-5) * lnw_ref[...] + lnb_ref[...]

        yt = jnp.tile(y, (1, NUM_GRIDS))
        gidx = jax.lax.broadcasted_iota(jnp.int32, (1, KAN_K), 1) // CODE
        gcols = GRID_MIN + gidx.astype(jnp.float32) * _DENOM
        tt = (yt - gcols) * (1.0 / _DENOM)
        rbf = jnp.exp(-(tt * tt))
        acc = _dot(_bf16(rbf), w3b_scr[...])
        out_ref[...] = jnp.maximum(acc + b3_ref[...], 0.0)


@jax.jit
def kernel(x, G, W1, b1, W2, b2, ln_w, ln_b, W3, b3):
    W3p = W3.reshape(CODE, NUM_GRIDS, D_OUT).transpose(1, 0, 2).reshape(KAN_K, D_OUT)
    row = lambda v: v.reshape(1, -1)

    full = lambda shape: pl.BlockSpec(shape, lambda t: (0,) * len(shape))

    code, feat_out = pl.pallas_call(
        _fused_kernel,
        grid=(T_TOTAL,),
        in_specs=[
            pl.BlockSpec((BM, N2), lambda t: (jnp.minimum(t, NA - 1), 0)),
            pl.BlockSpec((BM, N2), lambda t: (jnp.minimum(t, NA - 1), 1)),
            pl.BlockSpec((N2, D_IN), lambda t: (0, 0)),
            pl.BlockSpec((N2, D_IN), lambda t: (1, 0)),
            pl.BlockSpec((D_IN, B_HID // 2), lambda t: (0, 0)),
            pl.BlockSpec((D_IN, B_HID // 2), lambda t: (0, 1)),
            full((1, B_HID)),
            full((B_HID, CODE)),
            full((1, CODE)),
            full((1, CODE)),
            full((1, CODE)),
            full((KAN_K, D_OUT)),
            full((1, D_OUT)),
        ],
        out_specs=[
            pl.BlockSpec((BM, CODE), lambda t: (jnp.maximum(t - NA, 0), 0)),
            pl.BlockSpec((BM, D_OUT), lambda t: (jnp.maximum(t - NA, 0), 0)),
        ],
        out_shape=[
            jax.ShapeDtypeStruct((N, CODE), jnp.float32),
            jax.ShapeDtypeStruct((N, D_OUT), jnp.float32),
        ],
        scratch_shapes=[
            pltpu.VMEM((N, N), jnp.bfloat16),
            pltpu.VMEM((N, D_IN), jnp.bfloat16),
            pltpu.VMEM((D_IN, B_HID), jnp.bfloat16),
            pltpu.VMEM((B_HID, CODE), jnp.bfloat16),
            pltpu.VMEM((KAN_K, D_OUT), jnp.bfloat16),
            pltpu.VMEM((N, CODE), jnp.float32),
            pltpu.VMEM((N, CODE), jnp.float32),
        ],
        compiler_params=pltpu.CompilerParams(
            dimension_semantics=("arbitrary",)),
    )(G, G, x, x, W1, W1, row(b1), W2, row(b2), row(ln_w), row(ln_b), W3p, row(b3))
    return (code, feat_out)


# speedup vs baseline: 1.1747x; 1.0000x over previous
import jax
import jax.numpy as jnp
from jax.experimental import pallas as pl
from jax.experimental.pallas import tpu as pltpu

N = 2048
N2 = N // 2
N4 = N // 4
D_IN = 512
B_HID = 4096
CODE = 64
NUM_GRIDS = 8
GRID_MIN, GRID_MAX = -2.0, 2.0
D_OUT = 2 * D_IN
KAN_K = CODE * NUM_GRIDS

BM = 256
NA = N // BM
ND = N // BM
T_TOTAL = NA + ND
W3_CHUNK = D_OUT // NA

_DENOM = (GRID_MAX - GRID_MIN) / (NUM_GRIDS - 1)


def _dot(a, b):
    return jax.lax.dot_general(
        a, b, (((1,), (0,)), ((), ())),
        preferred_element_type=jnp.float32)


def _bf16(v):
    return v.astype(jnp.bfloat16)


def _fused_kernel(g0_ref, g1_ref, g2_ref, g3_ref, xt_ref, xb_ref, w1a_ref, w1b_ref,
                  b1_ref, w2_ref, b2_ref,
                  lnw_ref, lnb_ref, w3_ref, b3_ref,
                  code_ref, out_ref,
                  g_scr, xb_scr, w1b_scr, w2b_scr, w3b_scr, t2_scr, feat_scr):
    t = pl.program_id(0)

    @pl.when(t == 0)
    def _pack_resident():
        xb_scr[0:N2, :] = _bf16(xt_ref[...])
        xb_scr[N2:N, :] = _bf16(xb_ref[...])
        w1b_scr[:, 0:B_HID // 2] = _bf16(w1a_ref[...])
        w1b_scr[:, B_HID // 2:B_HID] = _bf16(w1b_ref[...])
        w2b_scr[...] = _bf16(w2_ref[...])

    @pl.when(t < NA)
    def _phase_a():
        i = t
        w3b_scr[:, pl.ds(i * W3_CHUNK, W3_CHUNK)] = \
            _bf16(w3_ref[:, pl.ds(i * W3_CHUNK, W3_CHUNK)])

        y = jnp.zeros((BM, D_IN), dtype=jnp.float32)
        for q, gq_ref in enumerate((g0_ref, g1_ref, g2_ref, g3_ref)):
            gqb = _bf16(gq_ref[...])
            g_scr[pl.ds(i * BM, BM), q * N4:(q + 1) * N4] = gqb
            y = y + _dot(gqb, xb_scr[q * N4:(q + 1) * N4, :])
        h = jnp.maximum(_dot(_bf16(y), w1b_scr[...]) + b1_ref[...], 0.0)
        t2_scr[pl.ds(i * BM, BM), :] = _dot(_bf16(h), w2b_scr[...])

    @pl.when(t >= NA)
    def _phase_d():
        i = t - NA

        @pl.when(i == 0)
        def _feat_all():
            t2b = _bf16(t2_scr[...])
            feat_scr[...] = _dot(g_scr[...], t2b)

        feat = feat_scr[pl.ds(i * BM, BM), :] + b2_ref[...]
        code = jnp.tanh(10.0 * feat)
        code_ref[...] = code

        mu = jnp.mean(code, axis=-1, keepdims=True)
        var = jnp.mean((code - mu) ** 2, axis=-1, keepdims=True)
        y = (code - mu) * jax.lax.rsqrt(var + 1e-5) * lnw_ref[...] + lnb_ref[...]

        yt = jnp.tile(y, (1, NUM_GRIDS))
        gidx = jax.lax.broadcasted_iota(jnp.int32, (1, KAN_K), 1) // CODE
        gcols = GRID_MIN + gidx.astype(jnp.float32) * _DENOM
        tt = (yt - gcols) * (1.0 / _DENOM)
        rbf = jnp.exp(-(tt * tt))
        acc = _dot(_bf16(rbf), w3b_scr[...])
        out_ref[...] = jnp.maximum(acc + b3_ref[...], 0.0)


@jax.jit
def kernel(x, G, W1, b1, W2, b2, ln_w, ln_b, W3, b3):
    W3p = W3.reshape(CODE, NUM_GRIDS, D_OUT).transpose(1, 0, 2).reshape(KAN_K, D_OUT)
    row = lambda v: v.reshape(1, -1)

    full = lambda shape: pl.BlockSpec(shape, lambda t: (0,) * len(shape))

    code, feat_out = pl.pallas_call(
        _fused_kernel,
        grid=(T_TOTAL,),
        in_specs=[
            pl.BlockSpec((BM, N4), lambda t: (jnp.minimum(t, NA - 1), 0)),
            pl.BlockSpec((BM, N4), lambda t: (jnp.minimum(t, NA - 1), 1)),
            pl.BlockSpec((BM, N4), lambda t: (jnp.minimum(t, NA - 1), 2)),
            pl.BlockSpec((BM, N4), lambda t: (jnp.minimum(t, NA - 1), 3)),
            pl.BlockSpec((N2, D_IN), lambda t: (0, 0)),
            pl.BlockSpec((N2, D_IN), lambda t: (1, 0)),
            pl.BlockSpec((D_IN, B_HID // 2), lambda t: (0, 0)),
            pl.BlockSpec((D_IN, B_HID // 2), lambda t: (0, 1)),
            full((1, B_HID)),
            full((B_HID, CODE)),
            full((1, CODE)),
            full((1, CODE)),
            full((1, CODE)),
            full((KAN_K, D_OUT)),
            full((1, D_OUT)),
        ],
        out_specs=[
            pl.BlockSpec((BM, CODE), lambda t: (jnp.maximum(t - NA, 0), 0)),
            pl.BlockSpec((BM, D_OUT), lambda t: (jnp.maximum(t - NA, 0), 0)),
        ],
        out_shape=[
            jax.ShapeDtypeStruct((N, CODE), jnp.float32),
            jax.ShapeDtypeStruct((N, D_OUT), jnp.float32),
        ],
        scratch_shapes=[
            pltpu.VMEM((N, N), jnp.bfloat16),
            pltpu.VMEM((N, D_IN), jnp.bfloat16),
            pltpu.VMEM((D_IN, B_HID), jnp.bfloat16),
            pltpu.VMEM((B_HID, CODE), jnp.bfloat16),
            pltpu.VMEM((KAN_K, D_OUT), jnp.bfloat16),
            pltpu.VMEM((N, CODE), jnp.float32),
            pltpu.VMEM((N, CODE), jnp.float32),
        ],
        compiler_params=pltpu.CompilerParams(
            dimension_semantics=("arbitrary",)),
    )(G, G, G, G, x, x, W1, W1, row(b1), W2, row(b2), row(ln_w), row(ln_b), W3p, row(b3))
    return (code, feat_out)
